# Initial kernel scaffold; baseline (speedup 1.0000x reference)
import jax
import jax.numpy as jnp
from jax.experimental import pallas as pl


def kernel(x, w1, b1, w2, b2, w3, b3, s1, s2, w_fc1, b_fc1, w_fc2, b_fc2):
    raise NotImplementedError("write your pallas kernel here")


# speedup vs baseline: 1.0871x; 1.0871x over previous
from functools import partial

import jax
import jax.numpy as jnp
from jax.experimental import pallas as pl
from jax.experimental.pallas import tpu as pltpu

_B_BLK = 8


def _conv3x3(a, offs, wcat, b):
    R, cin = a.shape
    L = R - offs[-1]
    cols = jnp.concatenate([a[o:o + L] for o in offs], axis=1)
    cols = jnp.concatenate(
        [cols, jnp.zeros((offs[-1], len(offs) * cin), a.dtype)], axis=0)
    z = jnp.dot(cols, wcat, preferred_element_type=jnp.float32) + b
    return jnp.maximum(z, 0.0)


def _pool2x2(z, d, C):
    R = z.shape[0]
    m = jnp.maximum(z[:R - d], z[d:])
    m = jnp.maximum(m[:-32], m[32:])
    m = jnp.concatenate([m, jnp.zeros((32 + d, C), z.dtype)], axis=0)
    m = m.reshape(-1, 2, 32, C)[:, 0]
    return m.reshape(R // 2, C)


def _fused_kernel(x_ref, w1_ref, b1_ref, w2_ref, b2_ref, w3_ref, b3_ref,
                  wf1_ref, bf1_ref, wf2_ref, bf2_ref, o_ref, *, Bb):
    a = x_ref[...].reshape(Bb * 1024, 8)
    w1 = w1_ref[...].reshape(72, 32)
    w2 = w2_ref[...].reshape(288, 64)
    w3 = w3_ref[...].reshape(576, 64)

    z1 = _conv3x3(a, [kh * 32 + kw for kh in range(3) for kw in range(3)],
                  w1, b1_ref[...])
    p1 = _pool2x2(z1, 1, 32)
    z2 = _conv3x3(p1, [kh * 32 + 2 * kw for kh in range(3) for kw in range(3)],
                  w2, b2_ref[...])
    p2 = _pool2x2(z2, 2, 64)
    z3 = _conv3x3(p2, [kh * 32 + 4 * kw for kh in range(3) for kw in range(3)],
                  w3, b3_ref[...])

    R = Bb * 256
    offs = [h * 32 + 4 * w for h in range(4) for w in range(4)]
    L = R - offs[-1]
    cols = jnp.concatenate([z3[o:o + L] for o in offs], axis=1)
    cols = jnp.concatenate(
        [cols, jnp.zeros((offs[-1], 1024), z3.dtype)], axis=0)
    h1 = jnp.maximum(
        jnp.dot(cols, wf1_ref[...], preferred_element_type=jnp.float32)
        + bf1_ref[...], 0.0)
    o = jnp.dot(h1, wf2_ref[...], preferred_element_type=jnp.float32) \
        + bf2_ref[...]
    o_ref[...] = o.reshape(Bb, 256, 10)[:, 0:1, :].astype(o_ref.dtype)


def _const_spec(shape):
    n = len(shape)
    return pl.BlockSpec(tuple(shape), lambda i, _n=n: (0,) * _n)


def kernel(x, w1, b1, w2, b2, w3, b3, s1, s2, w_fc1, b_fc1, w_fc2, b_fc2):
    del s1, s2
    B = x.shape[0]
    xr = jnp.transpose(x, (0, 2, 3, 1)).reshape(B, 1024, 3)
    xr = jnp.pad(xr, ((0, 0), (0, 0), (0, 5)))
    Bb = _B_BLK
    out = pl.pallas_call(
        partial(_fused_kernel, Bb=Bb),
        out_shape=jax.ShapeDtypeStruct((B, 1, 10), jnp.float32),
        grid=(B // Bb,),
        in_specs=[
            pl.BlockSpec((Bb, 1024, 8), lambda i: (i, 0, 0)),
            _const_spec(w1.shape), _const_spec(b1.shape),
            _const_spec(w2.shape), _const_spec(b2.shape),
            _const_spec(w3.shape), _const_spec(b3.shape),
            _const_spec(w_fc1.shape), _const_spec(b_fc1.shape),
            _const_spec(w_fc2.shape), _const_spec(b_fc2.shape),
        ],
        out_specs=pl.BlockSpec((Bb, 1, 10), lambda i: (i, 0, 0)),
        compiler_params=pltpu.CompilerParams(
            dimension_semantics=("parallel",)),
    )(xr, w1, b1, w2, b2, w3, b3, w_fc1, b_fc1, w_fc2, b_fc2)
    return out.reshape(B, 10)


# speedup vs baseline: 1.6874x; 1.5521x over previous
from functools import partial

import jax
import jax.numpy as jnp
from jax.experimental import pallas as pl
from jax.experimental.pallas import tpu as pltpu

_B_BLK = 8


def _conv3x3(a, offs, wcat, b):
    R, cin = a.shape
    L = R - offs[-1]
    cols = jnp.concatenate([a[o:o + L] for o in offs], axis=1)
    cols = jnp.concatenate(
        [cols, jnp.zeros((offs[-1], len(offs) * cin), a.dtype)], axis=0)
    z = jnp.dot(cols, wcat, preferred_element_type=jnp.float32) + b
    return jnp.maximum(z, 0.0)


def _pool2x2(z, d, C):
    R = z.shape[0]
    m = jnp.maximum(z[:R - d], z[d:])
    m = jnp.maximum(m[:-32], m[32:])
    m = jnp.concatenate([m, jnp.zeros((32 + d, C), z.dtype)], axis=0)
    m = m.reshape(-1, 2, 32, C)[:, 0]
    return m.reshape(R // 2, C)


def _fused_kernel(x_ref, w1_ref, b1_ref, w2_ref, b2_ref, w3_ref, b3_ref,
                  wf1_ref, bf1_ref, wf2_ref, bf2_ref, o_ref, *, Bb):
    a = jnp.transpose(x_ref[...], (0, 2, 1)).reshape(Bb * 1024, 3)
    w1 = w1_ref[...][:, :3, :].reshape(27, 32)
    w2 = w2_ref[...].reshape(288, 64)
    w3 = w3_ref[...].reshape(576, 64)

    z1 = _conv3x3(a, [kh * 32 + kw for kh in range(3) for kw in range(3)],
                  w1, b1_ref[...])
    p1 = _pool2x2(z1, 1, 32)
    z2 = _conv3x3(p1, [kh * 32 + 2 * kw for kh in range(3) for kw in range(3)],
                  w2, b2_ref[...])
    p2 = _pool2x2(z2, 2, 64)
    z3 = _conv3x3(p2, [kh * 32 + 4 * kw for kh in range(3) for kw in range(3)],
                  w3, b3_ref[...])

    R = Bb * 256
    offs = [h * 32 + 4 * w for h in range(4) for w in range(4)]
    L = R - offs[-1]
    cols = jnp.concatenate([z3[o:o + L] for o in offs], axis=1)
    cols = jnp.concatenate(
        [cols, jnp.zeros((offs[-1], 1024), z3.dtype)], axis=0)
    h1 = jnp.maximum(
        jnp.dot(cols, wf1_ref[...], preferred_element_type=jnp.float32)
        + bf1_ref[...], 0.0)
    o = jnp.dot(h1, wf2_ref[...], preferred_element_type=jnp.float32) \
        + bf2_ref[...]
    o_ref[...] = o.reshape(Bb, 256, 10)[:, 0:1, :].astype(o_ref.dtype)


def _const_spec(shape):
    n = len(shape)
    return pl.BlockSpec(tuple(shape), lambda i, _n=n: (0,) * _n)


def kernel(x, w1, b1, w2, b2, w3, b3, s1, s2, w_fc1, b_fc1, w_fc2, b_fc2):
    del s1, s2
    B = x.shape[0]
    xr = x.reshape(B, 3, 1024)
    Bb = _B_BLK
    out = pl.pallas_call(
        partial(_fused_kernel, Bb=Bb),
        out_shape=jax.ShapeDtypeStruct((B, 1, 10), jnp.float32),
        grid=(B // Bb,),
        in_specs=[
            pl.BlockSpec((Bb, 3, 1024), lambda i: (i, 0, 0)),
            _const_spec(w1.shape), _const_spec(b1.shape),
            _const_spec(w2.shape), _const_spec(b2.shape),
            _const_spec(w3.shape), _const_spec(b3.shape),
            _const_spec(w_fc1.shape), _const_spec(b_fc1.shape),
            _const_spec(w_fc2.shape), _const_spec(b_fc2.shape),
        ],
        out_specs=pl.BlockSpec((Bb, 1, 10), lambda i: (i, 0, 0)),
        compiler_params=pltpu.CompilerParams(
            dimension_semantics=("parallel",)),
    )(xr, w1, b1, w2, b2, w3, b3, w_fc1, b_fc1, w_fc2, b_fc2)
    return out.reshape(B, 10)


# speedup vs baseline: 2.8339x; 1.6795x over previous
from functools import partial

import jax
import jax.numpy as jnp
from jax.experimental import pallas as pl
from jax.experimental.pallas import tpu as pltpu

_B_BLK = 8
_DT = jnp.bfloat16


def _conv3x3(a, offs, wcat, b):
    R, cin = a.shape
    L = R - offs[-1]
    cols = jnp.concatenate([a[o:o + L] for o in offs], axis=1)
    cols = jnp.concatenate(
        [cols, jnp.zeros((offs[-1], len(offs) * cin), a.dtype)], axis=0)
    z = jnp.dot(cols, wcat.astype(_DT),
                preferred_element_type=jnp.float32) + b
    return jnp.maximum(z, 0.0).astype(_DT)


def _pool2x2(z, d, C):
    R = z.shape[0]
    m = jnp.maximum(z[:R - d], z[d:])
    m = jnp.maximum(m[:-32], m[32:])
    m = jnp.concatenate([m, jnp.zeros((32 + d, C), z.dtype)], axis=0)
    m = m.reshape(-1, 2, 32, C)[:, 0]
    return m.reshape(R // 2, C)


def _fused_kernel(x_ref, w1_ref, b1_ref, w2_ref, b2_ref, w3_ref, b3_ref,
                  wf1_ref, bf1_ref, wf2_ref, bf2_ref, o_ref, *, Bb):
    a = jnp.transpose(x_ref[...].astype(_DT), (0, 2, 1)).reshape(Bb * 1024, 3)
    w1 = w1_ref[...][:, :3, :].reshape(27, 32)
    w2 = w2_ref[...].reshape(288, 64)
    w3 = w3_ref[...].reshape(576, 64)

    z1 = _conv3x3(a, [kh * 32 + kw for kh in range(3) for kw in range(3)],
                  w1, b1_ref[...])
    p1 = _pool2x2(z1, 1, 32)
    z2 = _conv3x3(p1, [kh * 32 + 2 * kw for kh in range(3) for kw in range(3)],
                  w2, b2_ref[...])
    p2 = _pool2x2(z2, 2, 64)
    z3 = _conv3x3(p2, [kh * 32 + 4 * kw for kh in range(3) for kw in range(3)],
                  w3, b3_ref[...])

    R = Bb * 256
    offs = [h * 32 + 4 * w for h in range(4) for w in range(4)]
    L = R - offs[-1]
    cols = jnp.concatenate([z3[o:o + L] for o in offs], axis=1)
    cols = jnp.concatenate(
        [cols, jnp.zeros((offs[-1], 1024), z3.dtype)], axis=0)
    h1 = jnp.maximum(
        jnp.dot(cols, wf1_ref[...].astype(_DT),
                preferred_element_type=jnp.float32)
        + bf1_ref[...], 0.0).astype(_DT)
    o = jnp.dot(h1, wf2_ref[...].astype(_DT),
                preferred_element_type=jnp.float32) \
        + bf2_ref[...]
    o_ref[...] = o.reshape(Bb, 256, 10)[:, 0:1, :].astype(o_ref.dtype)


def _const_spec(shape):
    n = len(shape)
    return pl.BlockSpec(tuple(shape), lambda i, _n=n: (0,) * _n)


def kernel(x, w1, b1, w2, b2, w3, b3, s1, s2, w_fc1, b_fc1, w_fc2, b_fc2):
    del s1, s2
    B = x.shape[0]
    xr = x.reshape(B, 3, 1024)
    Bb = _B_BLK
    out = pl.pallas_call(
        partial(_fused_kernel, Bb=Bb),
        out_shape=jax.ShapeDtypeStruct((B, 1, 10), jnp.float32),
        grid=(B // Bb,),
        in_specs=[
            pl.BlockSpec((Bb, 3, 1024), lambda i: (i, 0, 0)),
            _const_spec(w1.shape), _const_spec(b1.shape),
            _const_spec(w2.shape), _const_spec(b2.shape),
            _const_spec(w3.shape), _const_spec(b3.shape),
            _const_spec(w_fc1.shape), _const_spec(b_fc1.shape),
            _const_spec(w_fc2.shape), _const_spec(b_fc2.shape),
        ],
        out_specs=pl.BlockSpec((Bb, 1, 10), lambda i: (i, 0, 0)),
        compiler_params=pltpu.CompilerParams(
            dimension_semantics=("parallel",)),
    )(xr, w1, b1, w2, b2, w3, b3, w_fc1, b_fc1, w_fc2, b_fc2)
    return out.reshape(B, 10)


# speedup vs baseline: 4.7814x; 1.6872x over previous
from functools import partial

import jax
import jax.numpy as jnp
from jax.experimental import pallas as pl
from jax.experimental.pallas import tpu as pltpu

_B_BLK = 8
_DT = jnp.bfloat16


def _conv3x3(a, offs, wcat, b):
    R, cin = a.shape
    L = R - offs[-1]
    cols = jnp.concatenate([a[o:o + L] for o in offs], axis=1)
    cols = jnp.concatenate(
        [cols, jnp.zeros((offs[-1], len(offs) * cin), a.dtype)], axis=0)
    z = jnp.dot(cols, wcat.astype(_DT),
                preferred_element_type=jnp.float32) + b
    return jnp.maximum(z, 0.0).astype(_DT)


def _pool2x2(z, d, C):
    R = z.shape[0]
    m = jnp.maximum(z[:R - d], z[d:])
    m = jnp.maximum(m[:-32], m[32:])
    m = jnp.concatenate([m, jnp.zeros((32 + d, C), z.dtype)], axis=0)
    m = m.reshape(-1, 2, 32, C)[:, 0]
    return m.reshape(R // 2, C)


def _fused_kernel(x_ref, w1g_ref, b1g_ref, w2_ref, b2_ref, w3_ref, b3_ref,
                  wf1_ref, bf1_ref, wf2_ref, bf2_ref, o_ref, *, Bb):
    w2 = w2_ref[...].reshape(288, 64)
    w3 = w3_ref[...].reshape(576, 64)

    t = jnp.transpose(x_ref[...].astype(_DT).reshape(Bb * 3, 1024), (1, 0))
    offs = [kh * 32 + kw for kh in range(3) for kw in range(3)]
    L = 1024 - offs[-1]
    cols = jnp.concatenate([t[o:o + L] for o in offs], axis=1)
    cols = jnp.concatenate(
        [cols, jnp.zeros((offs[-1], 27 * Bb), t.dtype)], axis=0)
    z1g = jnp.maximum(
        jnp.dot(cols, w1g_ref[...].astype(_DT),
                preferred_element_type=jnp.float32) + b1g_ref[...],
        0.0).astype(_DT)
    m = jnp.maximum(z1g[:-1], z1g[1:])
    m = jnp.maximum(m[:-32], m[32:])
    m = jnp.concatenate([m, jnp.zeros((33, Bb * 32), m.dtype)], axis=0)
    p1g = m.reshape(16, 2, 32, Bb * 32)[:, 0].reshape(512, Bb * 32)
    p1 = jnp.concatenate(
        [p1g[:, 32 * b:32 * (b + 1)] for b in range(Bb)], axis=0)
    z2 = _conv3x3(p1, [kh * 32 + 2 * kw for kh in range(3) for kw in range(3)],
                  w2, b2_ref[...])
    p2 = _pool2x2(z2, 2, 64)
    z3 = _conv3x3(p2, [kh * 32 + 4 * kw for kh in range(3) for kw in range(3)],
                  w3, b3_ref[...])

    R = Bb * 256
    offs = [h * 32 + 4 * w for h in range(4) for w in range(4)]
    L = R - offs[-1]
    cols = jnp.concatenate([z3[o:o + L] for o in offs], axis=1)
    cols = jnp.concatenate(
        [cols, jnp.zeros((offs[-1], 1024), z3.dtype)], axis=0)
    h1 = jnp.maximum(
        jnp.dot(cols, wf1_ref[...].astype(_DT),
                preferred_element_type=jnp.float32)
        + bf1_ref[...], 0.0).astype(_DT)
    o = jnp.dot(h1, wf2_ref[...].astype(_DT),
                preferred_element_type=jnp.float32) \
        + bf2_ref[...]
    o_ref[...] = o.reshape(Bb, 256, 10)[:, 0:1, :].astype(o_ref.dtype)


def _const_spec(shape):
    n = len(shape)
    return pl.BlockSpec(tuple(shape), lambda i, _n=n: (0,) * _n)


def kernel(x, w1, b1, w2, b2, w3, b3, s1, s2, w_fc1, b_fc1, w_fc2, b_fc2):
    del s1, s2
    B = x.shape[0]
    xr = x.reshape(B, 3, 1024)
    Bb = _B_BLK
    w1c = w1[:, :3, :]
    w1g = (w1c[:, None, :, None, :]
           * jnp.eye(Bb, dtype=w1.dtype)[None, :, None, :, None])
    w1g = w1g.reshape(9 * Bb * 3, Bb * 32)
    b1g = jnp.tile(b1, (1, Bb))
    out = pl.pallas_call(
        partial(_fused_kernel, Bb=Bb),
        out_shape=jax.ShapeDtypeStruct((B, 1, 10), jnp.float32),
        grid=(B // Bb,),
        in_specs=[
            pl.BlockSpec((Bb, 3, 1024), lambda i: (i, 0, 0)),
            _const_spec(w1g.shape), _const_spec(b1g.shape),
            _const_spec(w2.shape), _const_spec(b2.shape),
            _const_spec(w3.shape), _const_spec(b3.shape),
            _const_spec(w_fc1.shape), _const_spec(b_fc1.shape),
            _const_spec(w_fc2.shape), _const_spec(b_fc2.shape),
        ],
        out_specs=pl.BlockSpec((Bb, 1, 10), lambda i: (i, 0, 0)),
        compiler_params=pltpu.CompilerParams(
            dimension_semantics=("parallel",)),
    )(xr, w1g, b1g, w2, b2, w3, b3, w_fc1, b_fc1, w_fc2, b_fc2)
    return out.reshape(B, 10)


# speedup vs baseline: 5.6913x; 1.1903x over previous
from functools import partial

import jax
import jax.numpy as jnp
from jax.experimental import pallas as pl
from jax.experimental.pallas import tpu as pltpu

_B_BLK = 8
_DT = jnp.bfloat16


def _conv3x3(a, offs, wcat, b):
    R, cin = a.shape
    L = R - offs[-1]
    cols = jnp.concatenate([a[o:o + L] for o in offs], axis=1)
    cols = jnp.concatenate(
        [cols, jnp.zeros((offs[-1], len(offs) * cin), a.dtype)], axis=0)
    z = jnp.dot(cols, wcat.astype(_DT),
                preferred_element_type=jnp.float32) + b
    return jnp.maximum(z, 0.0).astype(_DT)


def _pool2x2(z, d, C):
    R = z.shape[0]
    m = jnp.maximum(z[:R - d], z[d:])
    m = jnp.maximum(m[:-32], m[32:])
    m = jnp.concatenate([m, jnp.zeros((32 + d, C), z.dtype)], axis=0)
    m = m.reshape(-1, 2, 32, C)[:, 0]
    return m.reshape(R // 2, C)


def _fused_kernel(x_ref, w1g_ref, b1g_ref, w2g_ref, b2g_ref, w3g_ref, b3g_ref,
                  wfg_ref, bfg_ref, wog_ref, bog_ref, o_ref, *, Bb):
    t = jnp.transpose(x_ref[...].astype(_DT).reshape(Bb * 3, 1024), (1, 0))
    offs = [kh * 32 + kw for kh in range(3) for kw in range(3)]
    L = 1024 - offs[-1]
    cols = jnp.concatenate([t[o:o + L] for o in offs], axis=1)
    cols = jnp.concatenate(
        [cols, jnp.zeros((offs[-1], 27 * Bb), t.dtype)], axis=0)
    z1g = jnp.maximum(
        jnp.dot(cols, w1g_ref[...].astype(_DT),
                preferred_element_type=jnp.float32) + b1g_ref[...],
        0.0).astype(_DT)
    m = jnp.maximum(z1g[:-1], z1g[1:])
    m = jnp.maximum(m[:-32], m[32:])
    m = jnp.concatenate([m, jnp.zeros((33, Bb * 32), m.dtype)], axis=0)
    p1g = m.reshape(16, 2, 32, Bb * 32)[:, 0].reshape(512, Bb * 32)

    a2 = jnp.concatenate([p1g[:, :128], p1g[:, 128:]], axis=0)

    z2 = _conv3x3(a2, [kh * 32 + 2 * kw for kh in range(3) for kw in range(3)],
                  w2g_ref[...], b2g_ref[...])
    p2 = _pool2x2(z2, 2, 256)
    z3 = _conv3x3(p2, [kh * 32 + 4 * kw for kh in range(3) for kw in range(3)],
                  w3g_ref[...], b3g_ref[...])

    wf = wfg_ref[...].astype(_DT)
    acc = bfg_ref[...].astype(jnp.float32)
    R = 512
    Lf = R - (3 * 32 + 4 * 3)
    for p in range(16):
        o = (p // 4) * 32 + (p % 4) * 4
        sl = z3[o:o + Lf]
        acc = acc + jnp.dot(sl, wf[p * 256:(p + 1) * 256],
                            preferred_element_type=jnp.float32)
    h1 = jnp.maximum(acc, 0.0).astype(_DT)
    out = jnp.dot(h1, wog_ref[...].astype(_DT),
                  preferred_element_type=jnp.float32) + bog_ref[...]
    out = jnp.concatenate(
        [out, jnp.zeros((R - Lf, 40), jnp.float32)], axis=0)
    o_ref[...] = out.reshape(2, 256, 40)[:, 0:1, :]


def _const_spec(shape):
    n = len(shape)
    return pl.BlockSpec(tuple(shape), lambda i, _n=n: (0,) * _n)


def _blockdiag(w, n):
    t, c, o = w.shape
    eye = jnp.eye(n, dtype=w.dtype)
    wg = w[:, None, :, None, :] * eye[None, :, None, :, None]
    return wg.reshape(t * n * c, n * o)


def kernel(x, w1, b1, w2, b2, w3, b3, s1, s2, w_fc1, b_fc1, w_fc2, b_fc2):
    del s1, s2
    B = x.shape[0]
    xr = x.reshape(B, 3, 1024)
    Bb = _B_BLK
    w1g = _blockdiag(w1[:, :3, :], Bb)
    b1g = jnp.tile(b1, (1, Bb))
    w2g = _blockdiag(w2, 4)
    b2g = jnp.tile(b2, (1, 4))
    w3g = _blockdiag(w3, 4)
    b3g = jnp.tile(b3, (1, 4))
    wfg = _blockdiag(w_fc1.reshape(16, 64, 64), 4)
    bfg = jnp.tile(b_fc1, (1, 4))
    wog = _blockdiag(w_fc2[None], 4)[:, :]
    bog = jnp.tile(b_fc2, (1, 4))
    out = pl.pallas_call(
        partial(_fused_kernel, Bb=Bb),
        out_shape=jax.ShapeDtypeStruct((B // 4, 1, 40), jnp.float32),
        grid=(B // Bb,),
        in_specs=[
            pl.BlockSpec((Bb, 3, 1024), lambda i: (i, 0, 0)),
            _const_spec(w1g.shape), _const_spec(b1g.shape),
            _const_spec(w2g.shape), _const_spec(b2g.shape),
            _const_spec(w3g.shape), _const_spec(b3g.shape),
            _const_spec(wfg.shape), _const_spec(bfg.shape),
            _const_spec(wog.shape), _const_spec(bog.shape),
        ],
        out_specs=pl.BlockSpec((2, 1, 40), lambda i: (i, 0, 0)),
        compiler_params=pltpu.CompilerParams(
            dimension_semantics=("parallel",)),
    )(xr, w1g, b1g, w2g, b2g, w3g, b3g, wfg, bfg, wog, bog)
    return out.reshape(B, 10)


# speedup vs baseline: 5.7117x; 1.0036x over previous
from functools import partial

import jax
import jax.numpy as jnp
from jax.experimental import pallas as pl
from jax.experimental.pallas import tpu as pltpu

_B_BLK = 16
_DT = jnp.bfloat16


def _conv3x3(a, offs, wcat, b):
    R, cin = a.shape
    L = R - offs[-1]
    cols = jnp.concatenate([a[o:o + L] for o in offs], axis=1)
    cols = jnp.concatenate(
        [cols, jnp.zeros((offs[-1], len(offs) * cin), a.dtype)], axis=0)
    z = jnp.dot(cols, wcat.astype(_DT),
                preferred_element_type=jnp.float32) + b
    return jnp.maximum(z, 0.0).astype(_DT)


def _pool2x2(z, d, C):
    R = z.shape[0]
    m = jnp.maximum(z[:R - d], z[d:])
    m = jnp.maximum(m[:-32], m[32:])
    m = jnp.concatenate([m, jnp.zeros((32 + d, C), z.dtype)], axis=0)
    m = m.reshape(-1, 2, 32, C)[:, 0]
    return m.reshape(R // 2, C)


def _fused_kernel(x_ref, w1g_ref, b1g_ref, w2g_ref, b2g_ref, w3g_ref, b3g_ref,
                  wfg_ref, bfg_ref, wog_ref, bog_ref, o_ref, *, Bb):
    t = jnp.transpose(x_ref[...].astype(_DT).reshape(Bb * 3, 1024), (1, 0))
    offs = [kh * 32 + kw for kh in range(3) for kw in range(3)]
    L = 1024 - offs[-1]
    cols = jnp.concatenate([t[o:o + L] for o in offs], axis=1)
    cols = jnp.concatenate(
        [cols, jnp.zeros((offs[-1], 27 * Bb), t.dtype)], axis=0)
    z1g = jnp.maximum(
        jnp.dot(cols, w1g_ref[...].astype(_DT),
                preferred_element_type=jnp.float32) + b1g_ref[...],
        0.0).astype(_DT)
    m = jnp.maximum(z1g[:-1], z1g[1:])
    m = jnp.maximum(m[:-32], m[32:])
    m = jnp.concatenate([m, jnp.zeros((33, Bb * 32), m.dtype)], axis=0)
    p1g = m.reshape(16, 2, 32, Bb * 32)[:, 0].reshape(512, Bb * 32)

    ng = Bb // 4
    a2 = jnp.concatenate(
        [p1g[:, 128 * j:128 * (j + 1)] for j in range(ng)], axis=0)

    z2 = _conv3x3(a2, [kh * 32 + 2 * kw for kh in range(3) for kw in range(3)],
                  w2g_ref[...], b2g_ref[...])
    p2 = _pool2x2(z2, 2, 256)
    z3 = _conv3x3(p2, [kh * 32 + 4 * kw for kh in range(3) for kw in range(3)],
                  w3g_ref[...], b3g_ref[...])

    wf = wfg_ref[...].astype(_DT)
    acc = bfg_ref[...].astype(jnp.float32)
    R = ng * 256
    Lf = R - (3 * 32 + 4 * 3)
    for p in range(16):
        o = (p // 4) * 32 + (p % 4) * 4
        sl = z3[o:o + Lf]
        acc = acc + jnp.dot(sl, wf[p * 256:(p + 1) * 256],
                            preferred_element_type=jnp.float32)
    h1 = jnp.maximum(acc, 0.0).astype(_DT)
    out = jnp.dot(h1, wog_ref[...].astype(_DT),
                  preferred_element_type=jnp.float32) + bog_ref[...]
    out = jnp.concatenate(
        [out, jnp.zeros((R - Lf, 40), jnp.float32)], axis=0)
    o_ref[...] = out.reshape(ng, 256, 40)[:, 0:1, :]


def _const_spec(shape):
    n = len(shape)
    return pl.BlockSpec(tuple(shape), lambda i, _n=n: (0,) * _n)


def _blockdiag(w, n):
    t, c, o = w.shape
    eye = jnp.eye(n, dtype=w.dtype)
    wg = w[:, None, :, None, :] * eye[None, :, None, :, None]
    return wg.reshape(t * n * c, n * o)


def kernel(x, w1, b1, w2, b2, w3, b3, s1, s2, w_fc1, b_fc1, w_fc2, b_fc2):
    del s1, s2
    B = x.shape[0]
    xr = x.reshape(B, 3, 1024)
    Bb = _B_BLK
    w1g = _blockdiag(w1[:, :3, :], Bb)
    b1g = jnp.tile(b1, (1, Bb))
    w2g = _blockdiag(w2, 4)
    b2g = jnp.tile(b2, (1, 4))
    w3g = _blockdiag(w3, 4)
    b3g = jnp.tile(b3, (1, 4))
    wfg = _blockdiag(w_fc1.reshape(16, 64, 64), 4)
    bfg = jnp.tile(b_fc1, (1, 4))
    wog = _blockdiag(w_fc2[None], 4)[:, :]
    bog = jnp.tile(b_fc2, (1, 4))
    out = pl.pallas_call(
        partial(_fused_kernel, Bb=Bb),
        out_shape=jax.ShapeDtypeStruct((B // 4, 1, 40), jnp.float32),
        grid=(B // Bb,),
        out_specs=pl.BlockSpec((Bb // 4, 1, 40), lambda i: (i, 0, 0)),
        in_specs=[
            pl.BlockSpec((Bb, 3, 1024), lambda i: (i, 0, 0)),
            _const_spec(w1g.shape), _const_spec(b1g.shape),
            _const_spec(w2g.shape), _const_spec(b2g.shape),
            _const_spec(w3g.shape), _const_spec(b3g.shape),
            _const_spec(wfg.shape), _const_spec(bfg.shape),
            _const_spec(wog.shape), _const_spec(bog.shape),
        ],
        compiler_params=pltpu.CompilerParams(
            dimension_semantics=("parallel",)),
    )(xr, w1g, b1g, w2g, b2g, w3g, b3g, wfg, bfg, wog, bog)
    return out.reshape(B, 10)
